# Initial kernel scaffold; baseline (speedup 1.0000x reference)
import jax
import jax.numpy as jnp
from jax.experimental import pallas as pl


def kernel(x, edge_index, edge_attr, W_msg, b_msg, W_upd, b_upd, ln_gamma, ln_beta):
    raise NotImplementedError("write your pallas kernel here")


# speedup vs baseline: 2.6853x; 2.6853x over previous
import functools

import jax
import jax.numpy as jnp
from jax import lax
from jax.experimental import pallas as pl
from jax.experimental.pallas import tpu as pltpu
from jax.experimental.pallas import tpu_sc as plsc

CB = 128
NODE_BLK = 1024


def _mm_bias_body(a_ref, w_ref, b_ref, o_ref):
    o_ref[...] = (
        jnp.dot(a_ref[...], w_ref[...], preferred_element_type=jnp.float32)
        + b_ref[...]
    )


def _mm_bias(a, w, b2, blk):
    rows, d_in = a.shape
    d_out = w.shape[1]
    return pl.pallas_call(
        _mm_bias_body,
        grid=(rows // blk,),
        in_specs=[
            pl.BlockSpec((blk, d_in), lambda i: (i, 0)),
            pl.BlockSpec((d_in, d_out), lambda i: (0, 0)),
            pl.BlockSpec((1, d_out), lambda i: (0, 0)),
        ],
        out_specs=pl.BlockSpec((blk, d_out), lambda i: (i, 0)),
        out_shape=jax.ShapeDtypeStruct((rows, d_out), jnp.float32),
    )(a, w, b2)


def _update_body(x_ref, agg_ref, cnt_ref, w1_ref, w2_ref, b_ref, g_ref,
                 bt_ref, o_ref):
    xb = x_ref[...]
    agg = agg_ref[0] + agg_ref[1]
    cnt8 = cnt_ref[0, 0] + cnt_ref[1, 0]
    nb = xb.shape[0]
    rows = lax.broadcasted_iota(jnp.int32, (nb, 8), 0) // 128
    sel = (rows == lax.broadcasted_iota(jnp.int32, (nb, 8), 1)).astype(
        jnp.float32)
    m = jnp.dot(sel, cnt8, preferred_element_type=jnp.float32)
    lanes = lax.broadcasted_iota(jnp.int32, (nb, 128), 0) % 128
    lmask = lanes == lax.broadcasted_iota(jnp.int32, (nb, 128), 1)
    c = jnp.sum(jnp.where(lmask, m, 0.0), axis=1, keepdims=True)
    mean = jnp.where(c > 0.0, agg / jnp.maximum(c, 1.0), 0.0)
    h = (
        jnp.dot(xb, w1_ref[...], preferred_element_type=jnp.float32)
        + jnp.dot(mean, w2_ref[...], preferred_element_type=jnp.float32)
        + b_ref[...]
    )
    mu = jnp.mean(h, axis=1, keepdims=True)
    d = h - mu
    var = jnp.mean(d * d, axis=1, keepdims=True)
    hn = d * lax.rsqrt(var + 1e-5) * g_ref[...] + bt_ref[...]
    o_ref[...] = jnp.maximum(hn + xb, 0.0)


@functools.lru_cache(maxsize=None)
def _make_sc_aggregate(nc, ns, k_chunks, n_pad, d):
    r = n_pad // ns
    nvec = d // 16
    mesh = plsc.VectorSubcoreMesh(core_axis_name="c", subcore_axis_name="s")

    @functools.partial(
        pl.kernel,
        out_type=(
            jax.ShapeDtypeStruct((nc, n_pad, d), jnp.float32),
            jax.ShapeDtypeStruct((nc, n_pad), jnp.float32),
        ),
        mesh=mesh,
        scratch_types=[
            pltpu.VMEM((1, CB), jnp.int32),
            pltpu.VMEM((1, CB), jnp.int32),
            pltpu.VMEM((CB, d), jnp.float32),
            pltpu.VMEM((CB, d), jnp.float32),
            pltpu.VMEM((CB,), jnp.float32),
            pltpu.VMEM((r,), jnp.float32),
            pltpu.VMEM_SHARED((n_pad, d), jnp.float32),
            pltpu.VMEM_SHARED((n_pad,), jnp.float32),
            pltpu.SemaphoreType.DMA,
        ],
    )
    def sc_fn(t_hbm, xw1_hbm, src_hbm, dst_hbm, agg_out, cnt_out,
              src_v, dst_v, t_v, gx_v, ones_v, czero_v,
              agg_sh, cnt_sh, sem):
        c = lax.axis_index("c")
        s = lax.axis_index("s")
        wid = c * ns + s

        def init_row(i, carry):
            for cc in range(nvec):
                t_v[i, pl.ds(cc * 16, 16)] = jnp.zeros((16,), jnp.float32)
            return carry

        lax.fori_loop(0, CB, init_row, 0)
        for cc in range(CB // 16):
            ones_v[pl.ds(cc * 16, 16)] = jnp.ones((16,), jnp.float32)

        def init_cz(i, carry):
            czero_v[pl.ds(i * 16, 16)] = jnp.zeros((16,), jnp.float32)
            return carry

        lax.fori_loop(0, r // 16, init_cz, 0)

        for kk in range(r // CB):
            pltpu.sync_copy(t_v, agg_sh.at[pl.ds(s * r + kk * CB, CB)])
        pltpu.sync_copy(czero_v, cnt_sh.at[pl.ds(s * r, r)])
        plsc.subcore_barrier()

        ebase = wid * (k_chunks * CB)

        def chunk(j, carry):
            pltpu.sync_copy(src_hbm.at[wid, pl.ds(j, 1)], src_v)
            pltpu.sync_copy(dst_hbm.at[wid, pl.ds(j, 1)], dst_v)
            pltpu.sync_copy(t_hbm.at[pl.ds(ebase + j * CB, CB)], t_v)
            pltpu.async_copy(xw1_hbm.at[src_v.at[0]], gx_v, sem).wait()

            def row(i, carry2):
                for cc in range(nvec):
                    sl = pl.ds(cc * 16, 16)
                    t_v[i, sl] = jnp.maximum(t_v[i, sl] + gx_v[i, sl], 0.0)
                return carry2

            lax.fori_loop(0, CB, row, 0)
            pltpu.sync_copy(t_v, agg_sh.at[dst_v.at[0]], add=True)
            pltpu.sync_copy(ones_v, cnt_sh.at[dst_v.at[0]], add=True)
            return carry

        lax.fori_loop(0, k_chunks, chunk, 0)
        plsc.subcore_barrier()

        pltpu.sync_copy(agg_sh.at[pl.ds(s * r, r)],
                        agg_out.at[c, pl.ds(s * r, r)])
        pltpu.sync_copy(cnt_sh.at[pl.ds(s * r, r)],
                        cnt_out.at[c, pl.ds(s * r, r)])

    return sc_fn


def kernel(x, edge_index, edge_attr, W_msg, b_msg, W_upd, b_upd, ln_gamma,
           ln_beta):
    n, d = x.shape
    e = edge_index.shape[1]
    d_out = W_msg.shape[1]
    f32 = jnp.float32

    info = plsc.get_sparse_core_info()
    nc, ns = info.num_cores, info.num_subcores
    nw = nc * ns

    k_chunks = -(-e // (nw * CB))
    e_pad = nw * k_chunks * CB
    align = max(NODE_BLK, ns * CB)
    n_pad = -(-(n + 1) // align) * align

    w1 = W_msg[:d]
    w2 = W_msg[d:]
    b2 = b_msg.reshape(1, d_out)
    zero_b = jnp.zeros((1, d_out), f32)

    x_pad = jnp.concatenate([x, jnp.zeros((n_pad - n, d), f32)], axis=0)
    xw1 = _mm_bias(x_pad, w1, zero_b, NODE_BLK)

    ea_pad = jnp.concatenate(
        [edge_attr, jnp.zeros((e_pad - e, d), f32)], axis=0)
    t = _mm_bias(ea_pad, w2, b2, 2048)

    src = jnp.concatenate(
        [edge_index[0], jnp.zeros((e_pad - e,), jnp.int32)])
    dst = jnp.concatenate(
        [edge_index[1], jnp.full((e_pad - e,), n, jnp.int32)])
    src3 = src.reshape(nw, k_chunks, CB)
    dst3 = dst.reshape(nw, k_chunks, CB)

    sc_fn = _make_sc_aggregate(nc, ns, k_chunks, n_pad, d_out)
    agg2, cnt2 = sc_fn(t, xw1, src3, dst3)
    cnt4 = cnt2.reshape(nc, n_pad // NODE_BLK, NODE_BLK // 128, 128)

    wu1 = W_upd[:d]
    wu2 = W_upd[d:]
    out = pl.pallas_call(
        _update_body,
        grid=(n_pad // NODE_BLK,),
        in_specs=[
            pl.BlockSpec((NODE_BLK, d), lambda i: (i, 0)),
            pl.BlockSpec((nc, NODE_BLK, d_out), lambda i: (0, i, 0)),
            pl.BlockSpec((nc, 1, NODE_BLK // 128, 128), lambda i: (0, i, 0, 0)),
            pl.BlockSpec((d, d_out), lambda i: (0, 0)),
            pl.BlockSpec((d_out, d_out), lambda i: (0, 0)),
            pl.BlockSpec((1, d_out), lambda i: (0, 0)),
            pl.BlockSpec((1, d_out), lambda i: (0, 0)),
            pl.BlockSpec((1, d_out), lambda i: (0, 0)),
        ],
        out_specs=pl.BlockSpec((NODE_BLK, d_out), lambda i: (i, 0)),
        out_shape=jax.ShapeDtypeStruct((n_pad, d_out), f32),
    )(x_pad, agg2, cnt4, wu1, wu2, b_upd.reshape(1, d_out),
      ln_gamma.reshape(1, d_out), ln_beta.reshape(1, d_out))

    return out[:n]


# speedup vs baseline: 3.2589x; 1.2136x over previous
import functools
import math

import jax
import jax.numpy as jnp
from jax import lax
from jax.experimental import pallas as pl
from jax.experimental.pallas import tpu as pltpu
from jax.experimental.pallas import tpu_sc as plsc

CB = 80
IB = 5
NODE_BLK = 1024


def _mm_bias_body(a_ref, w_ref, b_ref, o_ref):
    o_ref[...] = (
        jnp.dot(a_ref[...], w_ref[...], preferred_element_type=jnp.float32)
        + b_ref[...]
    )


def _mm_bias(a, w, b2, blk):
    rows, d_in = a.shape
    d_out = w.shape[1]
    return pl.pallas_call(
        _mm_bias_body,
        grid=(rows // blk,),
        in_specs=[
            pl.BlockSpec((blk, d_in), lambda i: (i, 0)),
            pl.BlockSpec((d_in, d_out), lambda i: (0, 0)),
            pl.BlockSpec((1, d_out), lambda i: (0, 0)),
        ],
        out_specs=pl.BlockSpec((blk, d_out), lambda i: (i, 0)),
        out_shape=jax.ShapeDtypeStruct((rows, d_out), jnp.float32),
    )(a, w, b2)


def _update_body(x_ref, agg_ref, cnt_ref, w1_ref, w2_ref, b_ref, g_ref,
                 bt_ref, o_ref):
    xb = x_ref[...]
    agg = agg_ref[0] + agg_ref[1]
    cnt8 = cnt_ref[0, 0] + cnt_ref[1, 0]
    nb = xb.shape[0]
    rows = lax.broadcasted_iota(jnp.int32, (nb, 8), 0) // 128
    sel = (rows == lax.broadcasted_iota(jnp.int32, (nb, 8), 1)).astype(
        jnp.float32)
    m = jnp.dot(sel, cnt8, preferred_element_type=jnp.float32)
    lanes = lax.broadcasted_iota(jnp.int32, (nb, 128), 0) % 128
    lmask = lanes == lax.broadcasted_iota(jnp.int32, (nb, 128), 1)
    c = jnp.sum(jnp.where(lmask, m, 0.0), axis=1, keepdims=True)
    mean = jnp.where(c > 0.0, agg / jnp.maximum(c, 1.0), 0.0)
    h = (
        jnp.dot(xb, w1_ref[...], preferred_element_type=jnp.float32)
        + jnp.dot(mean, w2_ref[...], preferred_element_type=jnp.float32)
        + b_ref[...]
    )
    mu = jnp.mean(h, axis=1, keepdims=True)
    d = h - mu
    var = jnp.mean(d * d, axis=1, keepdims=True)
    hn = d * lax.rsqrt(var + 1e-5) * g_ref[...] + bt_ref[...]
    o_ref[...] = jnp.maximum(hn + xb, 0.0)


@functools.lru_cache(maxsize=None)
def _make_sc_aggregate(nc, ns, k_chunks, n_pad, d):
    r = n_pad // ns
    nvec = d // 16
    mesh = plsc.VectorSubcoreMesh(core_axis_name="c", subcore_axis_name="s")

    @functools.partial(
        pl.kernel,
        out_type=(
            jax.ShapeDtypeStruct((nc, n_pad, d), jnp.float32),
            jax.ShapeDtypeStruct((nc, n_pad), jnp.float32),
        ),
        mesh=mesh,
        scratch_types=[
            pltpu.VMEM((2, 1, IB, CB), jnp.int32),
            pltpu.VMEM((2, 1, IB, CB), jnp.int32),
            pltpu.VMEM((CB, d), jnp.float32),
            pltpu.VMEM((CB, d), jnp.float32),
            pltpu.VMEM((CB, d), jnp.float32),
            pltpu.VMEM((CB, d), jnp.float32),
            pltpu.VMEM((CB,), jnp.float32),
            pltpu.VMEM((r,), jnp.float32),
            pltpu.VMEM_SHARED((n_pad, d), jnp.float32),
            pltpu.VMEM_SHARED((n_pad,), jnp.float32),
            pltpu.SemaphoreType.DMA,
            pltpu.SemaphoreType.DMA,
            pltpu.SemaphoreType.DMA,
            pltpu.SemaphoreType.DMA,
        ],
    )
    def sc_fn(t_hbm, xw1_hbm, src_hbm, dst_hbm, agg_out, cnt_out,
              src_v, dst_v, t_v0, t_v1, gx_v0, gx_v1, ones_v, czero_v,
              agg_sh, cnt_sh, sem_t0, sem_t1, sem_g0, sem_g1):
        c = lax.axis_index("c")
        s = lax.axis_index("s")
        wid = c * ns + s
        t_v = (t_v0, t_v1)
        gx_v = (gx_v0, gx_v1)
        sem_t = (sem_t0, sem_t1)
        sem_g = (sem_g0, sem_g1)
        ebase = wid * (k_chunks * CB)

        def init_row(i, carry):
            for cc in range(nvec):
                t_v0[i, pl.ds(cc * 16, 16)] = jnp.zeros((16,), jnp.float32)
            return carry

        lax.fori_loop(0, CB, init_row, 0)
        for cc in range(CB // 16):
            ones_v[pl.ds(cc * 16, 16)] = jnp.ones((16,), jnp.float32)

        def init_cz(i, carry):
            czero_v[pl.ds(i * 16, 16)] = jnp.zeros((16,), jnp.float32)
            return carry

        lax.fori_loop(0, r // 16, init_cz, 0)

        for kk in range(r // CB):
            pltpu.sync_copy(t_v0, agg_sh.at[pl.ds(s * r + kk * CB, CB)])
        pltpu.sync_copy(czero_v, cnt_sh.at[pl.ds(s * r, r)])
        plsc.subcore_barrier()

        def t_slice(j):
            return t_hbm.at[pl.ds(ebase + j * CB, CB)]

        def start(j, p):
            b = j // IB
            q = b % 2
            jm = j % IB

            @pl.when(jm == 0)
            def _():
                pltpu.sync_copy(src_hbm.at[wid, pl.ds(b, 1)], src_v.at[q])
                pltpu.sync_copy(dst_hbm.at[wid, pl.ds(b, 1)], dst_v.at[q])

            pltpu.async_copy(t_slice(j), t_v[p], sem_t[p])
            pltpu.async_copy(
                xw1_hbm.at[src_v.at[q, 0, jm]], gx_v[p], sem_g[p])

        def finish(j, p):
            q = (j // IB) % 2
            jm = j % IB
            pltpu.make_async_copy(t_slice(j), t_v[p], sem_t[p]).wait()
            pltpu.make_async_copy(
                xw1_hbm.at[src_v.at[q, 0, jm]], gx_v[p], sem_g[p]).wait()

            def row(i, carry2):
                for cc in range(nvec):
                    sl = pl.ds(cc * 16, 16)
                    t_v[p][i, sl] = jnp.maximum(
                        t_v[p][i, sl] + gx_v[p][i, sl], 0.0)
                return carry2

            lax.fori_loop(0, CB, row, 0, unroll=2)
            pltpu.sync_copy(
                t_v[p], agg_sh.at[dst_v.at[q, 0, jm]], add=True)
            pltpu.sync_copy(
                ones_v, cnt_sh.at[dst_v.at[q, 0, jm]], add=True)

        start(0, 0)

        def pair(g, carry):
            j = 2 * g
            start(j + 1, 1)
            finish(j, 0)
            start(j + 2, 0)
            finish(j + 1, 1)
            return carry

        lax.fori_loop(0, (k_chunks - 1) // 2, pair, 0)
        finish(k_chunks - 1, 0)
        plsc.subcore_barrier()

        pltpu.sync_copy(agg_sh.at[pl.ds(s * r, r)],
                        agg_out.at[c, pl.ds(s * r, r)])
        pltpu.sync_copy(cnt_sh.at[pl.ds(s * r, r)],
                        cnt_out.at[c, pl.ds(s * r, r)])

    return sc_fn


def kernel(x, edge_index, edge_attr, W_msg, b_msg, W_upd, b_upd, ln_gamma,
           ln_beta):
    n, d = x.shape
    e = edge_index.shape[1]
    d_out = W_msg.shape[1]
    f32 = jnp.float32

    info = plsc.get_sparse_core_info()
    nc, ns = info.num_cores, info.num_subcores
    nw = nc * ns

    k_chunks = -(-e // (nw * CB))
    while k_chunks % 2 == 0 or k_chunks % IB != 0:
        k_chunks += 1
    e_pad = nw * k_chunks * CB
    align = math.lcm(NODE_BLK, ns * CB)
    n_pad = -(-(n + 1) // align) * align

    w1 = W_msg[:d]
    w2 = W_msg[d:]
    b2 = b_msg.reshape(1, d_out)
    zero_b = jnp.zeros((1, d_out), f32)

    x_pad = jnp.concatenate([x, jnp.zeros((n_pad - n, d), f32)], axis=0)
    xw1 = _mm_bias(x_pad, w1, zero_b, NODE_BLK)

    if e_pad > e:
        ea_pad = jnp.concatenate(
            [edge_attr, jnp.zeros((e_pad - e, d), f32)], axis=0)
        src = jnp.concatenate(
            [edge_index[0], jnp.zeros((e_pad - e,), jnp.int32)])
        dst = jnp.concatenate(
            [edge_index[1], jnp.full((e_pad - e,), n, jnp.int32)])
    else:
        ea_pad = edge_attr
        src = edge_index[0]
        dst = edge_index[1]
    t = _mm_bias(ea_pad, w2, b2, nw * CB)
    src3 = src.reshape(nw, k_chunks // IB, IB, CB)
    dst3 = dst.reshape(nw, k_chunks // IB, IB, CB)

    sc_fn = _make_sc_aggregate(nc, ns, k_chunks, n_pad, d_out)
    agg2, cnt2 = sc_fn(t, xw1, src3, dst3)
    cnt4 = cnt2.reshape(nc, n_pad // NODE_BLK, NODE_BLK // 128, 128)

    wu1 = W_upd[:d]
    wu2 = W_upd[d:]
    out = pl.pallas_call(
        _update_body,
        grid=(n_pad // NODE_BLK,),
        in_specs=[
            pl.BlockSpec((NODE_BLK, d), lambda i: (i, 0)),
            pl.BlockSpec((nc, NODE_BLK, d_out), lambda i: (0, i, 0)),
            pl.BlockSpec((nc, 1, NODE_BLK // 128, 128), lambda i: (0, i, 0, 0)),
            pl.BlockSpec((d, d_out), lambda i: (0, 0)),
            pl.BlockSpec((d_out, d_out), lambda i: (0, 0)),
            pl.BlockSpec((1, d_out), lambda i: (0, 0)),
            pl.BlockSpec((1, d_out), lambda i: (0, 0)),
            pl.BlockSpec((1, d_out), lambda i: (0, 0)),
        ],
        out_specs=pl.BlockSpec((NODE_BLK, d_out), lambda i: (i, 0)),
        out_shape=jax.ShapeDtypeStruct((n_pad, d_out), f32),
    )(x_pad, agg2, cnt4, wu1, wu2, b_upd.reshape(1, d_out),
      ln_gamma.reshape(1, d_out), ln_beta.reshape(1, d_out))

    return out[:n]
